# baseline (speedup 1.0000x reference)
import functools

import jax
import jax.numpy as jnp
from jax import lax
from jax.experimental import pallas as pl
from jax.experimental.pallas import tpu as pltpu
from jax.experimental.pallas import tpu_sc as plsc

N = 10000
E = 320000
D = 128
D2 = 64
NCLS = 40

NC = 2
NS = 16
CHUNK = 128
CH0 = 96
CH1 = 64
CHMAX = max(CH0, CH1)
G = 4
E_PAD = NS * (CH0 + CH1) * CHUNK
ZROWS = 640
ZBLK = 64
ACC_ROWS = NS * ZROWS
ROWS_OUT = 1000

_mesh = plsc.VectorSubcoreMesh(core_axis_name="c", subcore_axis_name="s")


def _make_sc_aggregate(d):
    @functools.partial(
        pl.kernel,
        mesh=_mesh,
        compiler_params=pltpu.CompilerParams(
            use_tc_tiling_on_sc=(d == D)),
        out_type=jax.ShapeDtypeStruct((NC, N, d), jnp.float32),
        scratch_types=[
            pltpu.VMEM((G, CHUNK), jnp.int32),
            pltpu.VMEM((G, CHUNK), jnp.int32),
            pltpu.VMEM((G, CHUNK), jnp.int32),
            pltpu.VMEM((G, CHUNK), jnp.int32),
            pltpu.VMEM((CHUNK, d), jnp.float32),
            pltpu.VMEM((CHUNK, d), jnp.float32),
            pltpu.SemaphoreType.DMA,
            pltpu.SemaphoreType.DMA,
            pltpu.SemaphoreType.DMA,
            pltpu.SemaphoreType.DMA,
            pltpu.VMEM_SHARED((ACC_ROWS, d), jnp.float32),
        ],
    )
    def sc_aggregate(h_hbm, srcs_hbm, dsts_hbm, zeros_hbm, out_hbm,
                     srcga, dstga, srcgb, dstgb, buf0, buf1,
                     semia, semib, semg0, semg1, acc):
        c = lax.axis_index("c")
        s = lax.axis_index("s")
        base = jnp.where(c == 0, s * CH0, NS * CH0 + s * CH1)
        nch = jnp.where(c == 0, CH0, CH1)

        bufs = (buf0, buf1)
        semg = (semg0, semg1)

        def idx_fill(grp_first_chunk, srcg, dstg, sem):
            pltpu.async_copy(srcs_hbm.at[pl.ds(base + grp_first_chunk, G)],
                             srcg, sem)
            pltpu.async_copy(dsts_hbm.at[pl.ds(base + grp_first_chunk, G)],
                             dstg, sem)

        def idx_wait(srcg, dstg, sem):
            pltpu.make_async_copy(srcs_hbm.at[pl.ds(0, G)], srcg, sem).wait()
            pltpu.make_async_copy(dsts_hbm.at[pl.ds(0, G)], dstg, sem).wait()

        def gather_start(srcg, k, b):
            pltpu.async_copy(h_hbm.at[srcg.at[k]], bufs[b], semg[b])

        def gather_wait(b):
            pltpu.make_async_copy(h_hbm.at[pl.ds(0, CHUNK)], bufs[b],
                                  semg[b]).wait()

        for k in range(ZROWS // ZBLK):
            pltpu.sync_copy(zeros_hbm,
                            acc.at[pl.ds(s * ZROWS + k * ZBLK, ZBLK)])
        plsc.subcore_barrier()

        @pl.when(nch > 0)
        def _():
            pltpu.sync_copy(srcs_hbm.at[pl.ds(base, G)], srcga)
            pltpu.sync_copy(dsts_hbm.at[pl.ds(base, G)], dstga)
            idx_fill(G, srcgb, dstgb, semib)
            gather_start(srcga, 0, 0)

        def body(u, carry):
            j0 = 8 * u

            @pl.when(j0 < nch)
            def _():
                for k in range(8):
                    grp_cur = (srcga, dstga) if k < 4 else (srcgb, dstgb)
                    b = k % 2
                    nb = (k + 1) % 2
                    if k < 3:
                        gather_start(grp_cur[0], k + 1, nb)
                    elif k == 3:
                        idx_wait(srcgb, dstgb, semib)
                        gather_start(srcgb, 0, nb)
                    elif k < 7:
                        gather_start(srcgb, k - 3, nb)
                    else:
                        @pl.when(j0 + 8 < nch)
                        def _():
                            idx_wait(srcga, dstga, semia)
                            gather_start(srcga, 0, nb)
                    gather_wait(b)
                    pltpu.sync_copy(bufs[b], acc.at[grp_cur[1].at[k % 4]],
                                    add=True)
                    if k == 3:
                        @pl.when(j0 + 8 < nch)
                        def _():
                            idx_fill(j0 + 8, srcga, dstga, semia)

                @pl.when(j0 + 12 < nch)
                def _():
                    idx_fill(j0 + 12, srcgb, dstgb, semib)

            return carry

        lax.fori_loop(0, CHMAX // 8, body, 0)

        plsc.subcore_barrier()

        @pl.when(s < N // ROWS_OUT)
        def _():
            pltpu.sync_copy(acc.at[pl.ds(s * ROWS_OUT, ROWS_OUT)],
                            out_hbm.at[c, pl.ds(s * ROWS_OUT, ROWS_OUT)])

    return sc_aggregate


_sc_aggregate_d = _make_sc_aggregate(D)
_sc_aggregate_d2 = _make_sc_aggregate(D2)


BR = 1000


def _mm_body(x_ref, w_ref, o_ref):
    o_ref[...] = jnp.dot(x_ref[...], w_ref[...],
                         preferred_element_type=jnp.float32)


def _fuse_body(p_ref, b_ref, w_ref, o_ref):
    h = jnp.maximum(p_ref[0] + p_ref[1] + b_ref[...], 0.0)
    o_ref[...] = jnp.dot(h, w_ref[...], preferred_element_type=jnp.float32)


def _bias_body(p_ref, b_ref, o_ref):
    o_ref[...] = p_ref[0] + p_ref[1] + b_ref[...]


def _tc_matmul(x, w):
    return pl.pallas_call(
        _mm_body,
        grid=(N // BR,),
        in_specs=[pl.BlockSpec((BR, D), lambda i: (i, 0)),
                  pl.BlockSpec((D, D), lambda i: (0, 0))],
        out_specs=pl.BlockSpec((BR, D), lambda i: (i, 0)),
        out_shape=jax.ShapeDtypeStruct((N, D), jnp.float32),
    )(x, w)


def _tc_fused(p, b, w, dout):
    return pl.pallas_call(
        _fuse_body,
        grid=(N // BR,),
        in_specs=[pl.BlockSpec((2, BR, D), lambda i: (0, i, 0)),
                  pl.BlockSpec((1, D), lambda i: (0, 0)),
                  pl.BlockSpec((D, dout), lambda i: (0, 0))],
        out_specs=pl.BlockSpec((BR, dout), lambda i: (i, 0)),
        out_shape=jax.ShapeDtypeStruct((N, dout), jnp.float32),
    )(p, b, w)


def _tc_bias(p, b, dout):
    return pl.pallas_call(
        _bias_body,
        grid=(N // BR,),
        in_specs=[pl.BlockSpec((2, BR, dout), lambda i: (0, i, 0)),
                  pl.BlockSpec((1, dout), lambda i: (0, 0))],
        out_specs=pl.BlockSpec((BR, dout), lambda i: (i, 0)),
        out_shape=jax.ShapeDtypeStruct((N, dout), jnp.float32),
    )(p, b)


def kernel(features, edge_index, W0, b0, W1, b1, W2, b2):
    src = edge_index[0]
    dst = edge_index[1]
    pad = E_PAD - E
    srcs = jnp.concatenate(
        [src, jnp.zeros((pad,), jnp.int32)]).reshape(-1, CHUNK)
    dsts = jnp.concatenate(
        [dst, jnp.full((pad,), ACC_ROWS - 1, jnp.int32)]).reshape(-1, CHUNK)
    zeros = jnp.zeros((ZBLK, D), jnp.float32)
    zeros2 = jnp.zeros((ZBLK, D2), jnp.float32)

    a = _tc_matmul(features, W0)
    p = _sc_aggregate_d(a, srcs, dsts, zeros)
    c = _tc_fused(p, b0.reshape(1, D), W1, D)
    q = _sc_aggregate_d(c, srcs, dsts, zeros)
    w2p = jnp.pad(W2, ((0, 0), (0, D2 - NCLS)))
    h2 = _tc_fused(q, b1.reshape(1, D), w2p, D2)
    r = _sc_aggregate_d2(h2, srcs, dsts, zeros2)
    b2p = jnp.pad(b2, (0, D2 - NCLS)).reshape(1, D2)
    o = _tc_bias(r, b2p, D2)
    return o[:, :NCLS]


# speedup vs baseline: 1.0257x; 1.0257x over previous
import functools

import jax
import jax.numpy as jnp
from jax import lax
from jax.experimental import pallas as pl
from jax.experimental.pallas import tpu as pltpu
from jax.experimental.pallas import tpu_sc as plsc

N = 10000
E = 320000
D = 128
D2 = 64
NCLS = 40

NC = 2
NS = 16
CHUNK = 128
CH0 = 120
CH1 = 40
CHMAX = max(CH0, CH1)
G = 4
E_PAD = NS * (CH0 + CH1) * CHUNK
ZROWS = 640
ZBLK = 64
ACC_ROWS = NS * ZROWS
ROWS_OUT = 1000

_mesh = plsc.VectorSubcoreMesh(core_axis_name="c", subcore_axis_name="s")


def _make_sc_aggregate(d):
    @functools.partial(
        pl.kernel,
        mesh=_mesh,
        compiler_params=pltpu.CompilerParams(
            use_tc_tiling_on_sc=(d == D)),
        out_type=jax.ShapeDtypeStruct((NC, N, d), jnp.float32),
        scratch_types=[
            pltpu.VMEM((G, CHUNK), jnp.int32),
            pltpu.VMEM((G, CHUNK), jnp.int32),
            pltpu.VMEM((G, CHUNK), jnp.int32),
            pltpu.VMEM((G, CHUNK), jnp.int32),
            pltpu.VMEM((CHUNK, d), jnp.float32),
            pltpu.VMEM((CHUNK, d), jnp.float32),
            pltpu.SemaphoreType.DMA,
            pltpu.SemaphoreType.DMA,
            pltpu.SemaphoreType.DMA,
            pltpu.SemaphoreType.DMA,
            pltpu.VMEM_SHARED((ACC_ROWS, d), jnp.float32),
        ],
    )
    def sc_aggregate(h_hbm, srcs_hbm, dsts_hbm, zeros_hbm, out_hbm,
                     srcga, dstga, srcgb, dstgb, buf0, buf1,
                     semia, semib, semg0, semg1, acc):
        c = lax.axis_index("c")
        s = lax.axis_index("s")
        base = jnp.where(c == 0, s * CH0, NS * CH0 + s * CH1)
        nch = jnp.where(c == 0, CH0, CH1)

        bufs = (buf0, buf1)
        semg = (semg0, semg1)

        def idx_fill(grp_first_chunk, srcg, dstg, sem):
            pltpu.async_copy(srcs_hbm.at[pl.ds(base + grp_first_chunk, G)],
                             srcg, sem)
            pltpu.async_copy(dsts_hbm.at[pl.ds(base + grp_first_chunk, G)],
                             dstg, sem)

        def idx_wait(srcg, dstg, sem):
            pltpu.make_async_copy(srcs_hbm.at[pl.ds(0, G)], srcg, sem).wait()
            pltpu.make_async_copy(dsts_hbm.at[pl.ds(0, G)], dstg, sem).wait()

        def gather_start(srcg, k, b):
            pltpu.async_copy(h_hbm.at[srcg.at[k]], bufs[b], semg[b])

        def gather_wait(b):
            pltpu.make_async_copy(h_hbm.at[pl.ds(0, CHUNK)], bufs[b],
                                  semg[b]).wait()

        for k in range(ZROWS // ZBLK):
            pltpu.sync_copy(zeros_hbm,
                            acc.at[pl.ds(s * ZROWS + k * ZBLK, ZBLK)])
        plsc.subcore_barrier()

        @pl.when(nch > 0)
        def _():
            pltpu.sync_copy(srcs_hbm.at[pl.ds(base, G)], srcga)
            pltpu.sync_copy(dsts_hbm.at[pl.ds(base, G)], dstga)
            idx_fill(G, srcgb, dstgb, semib)
            gather_start(srcga, 0, 0)

        def body(u, carry):
            j0 = 8 * u

            @pl.when(j0 < nch)
            def _():
                for k in range(8):
                    grp_cur = (srcga, dstga) if k < 4 else (srcgb, dstgb)
                    b = k % 2
                    nb = (k + 1) % 2
                    if k < 3:
                        gather_start(grp_cur[0], k + 1, nb)
                    elif k == 3:
                        idx_wait(srcgb, dstgb, semib)
                        gather_start(srcgb, 0, nb)
                    elif k < 7:
                        gather_start(srcgb, k - 3, nb)
                    else:
                        @pl.when(j0 + 8 < nch)
                        def _():
                            idx_wait(srcga, dstga, semia)
                            gather_start(srcga, 0, nb)
                    gather_wait(b)
                    pltpu.sync_copy(bufs[b], acc.at[grp_cur[1].at[k % 4]],
                                    add=True)
                    if k == 3:
                        @pl.when(j0 + 8 < nch)
                        def _():
                            idx_fill(j0 + 8, srcga, dstga, semia)

                @pl.when(j0 + 12 < nch)
                def _():
                    idx_fill(j0 + 12, srcgb, dstgb, semib)

            return carry

        lax.fori_loop(0, CHMAX // 8, body, 0)

        plsc.subcore_barrier()

        @pl.when(s < N // ROWS_OUT)
        def _():
            pltpu.sync_copy(acc.at[pl.ds(s * ROWS_OUT, ROWS_OUT)],
                            out_hbm.at[c, pl.ds(s * ROWS_OUT, ROWS_OUT)])

    return sc_aggregate


_sc_aggregate_d = _make_sc_aggregate(D)
_sc_aggregate_d2 = _make_sc_aggregate(D2)


BR = 1000


def _mm_body(x_ref, w_ref, o_ref):
    o_ref[...] = jnp.dot(x_ref[...], w_ref[...],
                         preferred_element_type=jnp.float32)


def _fuse_body(p_ref, b_ref, w_ref, o_ref):
    h = jnp.maximum(p_ref[0] + p_ref[1] + b_ref[...], 0.0)
    o_ref[...] = jnp.dot(h, w_ref[...], preferred_element_type=jnp.float32)


def _bias_body(p_ref, b_ref, o_ref):
    o_ref[...] = p_ref[0] + p_ref[1] + b_ref[...]


def _tc_matmul(x, w):
    return pl.pallas_call(
        _mm_body,
        grid=(N // BR,),
        in_specs=[pl.BlockSpec((BR, D), lambda i: (i, 0)),
                  pl.BlockSpec((D, D), lambda i: (0, 0))],
        out_specs=pl.BlockSpec((BR, D), lambda i: (i, 0)),
        out_shape=jax.ShapeDtypeStruct((N, D), jnp.float32),
    )(x, w)


def _tc_fused(p, b, w, dout):
    return pl.pallas_call(
        _fuse_body,
        grid=(N // BR,),
        in_specs=[pl.BlockSpec((2, BR, D), lambda i: (0, i, 0)),
                  pl.BlockSpec((1, D), lambda i: (0, 0)),
                  pl.BlockSpec((D, dout), lambda i: (0, 0))],
        out_specs=pl.BlockSpec((BR, dout), lambda i: (i, 0)),
        out_shape=jax.ShapeDtypeStruct((N, dout), jnp.float32),
    )(p, b, w)


def _tc_bias(p, b, dout):
    return pl.pallas_call(
        _bias_body,
        grid=(N // BR,),
        in_specs=[pl.BlockSpec((2, BR, dout), lambda i: (0, i, 0)),
                  pl.BlockSpec((1, dout), lambda i: (0, 0))],
        out_specs=pl.BlockSpec((BR, dout), lambda i: (i, 0)),
        out_shape=jax.ShapeDtypeStruct((N, dout), jnp.float32),
    )(p, b)


def kernel(features, edge_index, W0, b0, W1, b1, W2, b2):
    src = edge_index[0]
    dst = edge_index[1]
    pad = E_PAD - E
    srcs = jnp.concatenate(
        [src, jnp.zeros((pad,), jnp.int32)]).reshape(-1, CHUNK)
    dsts = jnp.concatenate(
        [dst, jnp.full((pad,), ACC_ROWS - 1, jnp.int32)]).reshape(-1, CHUNK)
    zeros = jnp.zeros((ZBLK, D), jnp.float32)
    zeros2 = jnp.zeros((ZBLK, D2), jnp.float32)

    a = _tc_matmul(features, W0)
    p = _sc_aggregate_d(a, srcs, dsts, zeros)
    c = _tc_fused(p, b0.reshape(1, D), W1, D)
    q = _sc_aggregate_d(c, srcs, dsts, zeros)
    w2p = jnp.pad(W2, ((0, 0), (0, D2 - NCLS)))
    h2 = _tc_fused(q, b1.reshape(1, D), w2p, D2)
    r = _sc_aggregate_d2(h2, srcs, dsts, zeros2)
    b2p = jnp.pad(b2, (0, D2 - NCLS)).reshape(1, D2)
    o = _tc_bias(r, b2p, D2)
    return o[:, :NCLS]


# speedup vs baseline: 1.0313x; 1.0055x over previous
import functools

import jax
import jax.numpy as jnp
from jax import lax
from jax.experimental import pallas as pl
from jax.experimental.pallas import tpu as pltpu
from jax.experimental.pallas import tpu_sc as plsc

N = 10000
E = 320000
D = 128
D2 = 64
NCLS = 40

NC = 2
NS = 16
CHUNK = 128
CH0 = 128
CH1 = 32
CHMAX = max(CH0, CH1)
G = 4
E_PAD = NS * (CH0 + CH1) * CHUNK
ZROWS = 640
ZBLK = 64
ACC_ROWS = NS * ZROWS
ROWS_OUT = 1000

_mesh = plsc.VectorSubcoreMesh(core_axis_name="c", subcore_axis_name="s")


def _make_sc_aggregate(d):
    @functools.partial(
        pl.kernel,
        mesh=_mesh,
        compiler_params=pltpu.CompilerParams(
            use_tc_tiling_on_sc=(d == D)),
        out_type=jax.ShapeDtypeStruct((NC, N, d), jnp.float32),
        scratch_types=[
            pltpu.VMEM((G, CHUNK), jnp.int32),
            pltpu.VMEM((G, CHUNK), jnp.int32),
            pltpu.VMEM((G, CHUNK), jnp.int32),
            pltpu.VMEM((G, CHUNK), jnp.int32),
            pltpu.VMEM((CHUNK, d), jnp.float32),
            pltpu.VMEM((CHUNK, d), jnp.float32),
            pltpu.SemaphoreType.DMA,
            pltpu.SemaphoreType.DMA,
            pltpu.SemaphoreType.DMA,
            pltpu.SemaphoreType.DMA,
            pltpu.VMEM_SHARED((ACC_ROWS, d), jnp.float32),
        ],
    )
    def sc_aggregate(h_hbm, srcs_hbm, dsts_hbm, zeros_hbm, out_hbm,
                     srcga, dstga, srcgb, dstgb, buf0, buf1,
                     semia, semib, semg0, semg1, acc):
        c = lax.axis_index("c")
        s = lax.axis_index("s")
        base = jnp.where(c == 0, s * CH0, NS * CH0 + s * CH1)
        nch = jnp.where(c == 0, CH0, CH1)

        bufs = (buf0, buf1)
        semg = (semg0, semg1)

        def idx_fill(grp_first_chunk, srcg, dstg, sem):
            pltpu.async_copy(srcs_hbm.at[pl.ds(base + grp_first_chunk, G)],
                             srcg, sem)
            pltpu.async_copy(dsts_hbm.at[pl.ds(base + grp_first_chunk, G)],
                             dstg, sem)

        def idx_wait(srcg, dstg, sem):
            pltpu.make_async_copy(srcs_hbm.at[pl.ds(0, G)], srcg, sem).wait()
            pltpu.make_async_copy(dsts_hbm.at[pl.ds(0, G)], dstg, sem).wait()

        def gather_start(srcg, k, b):
            pltpu.async_copy(h_hbm.at[srcg.at[k]], bufs[b], semg[b])

        def gather_wait(b):
            pltpu.make_async_copy(h_hbm.at[pl.ds(0, CHUNK)], bufs[b],
                                  semg[b]).wait()

        for k in range(ZROWS // ZBLK):
            pltpu.sync_copy(zeros_hbm,
                            acc.at[pl.ds(s * ZROWS + k * ZBLK, ZBLK)])
        plsc.subcore_barrier()

        @pl.when(nch > 0)
        def _():
            pltpu.sync_copy(srcs_hbm.at[pl.ds(base, G)], srcga)
            pltpu.sync_copy(dsts_hbm.at[pl.ds(base, G)], dstga)
            idx_fill(G, srcgb, dstgb, semib)
            gather_start(srcga, 0, 0)

        def body(u, carry):
            j0 = 8 * u

            @pl.when(j0 < nch)
            def _():
                for k in range(8):
                    grp_cur = (srcga, dstga) if k < 4 else (srcgb, dstgb)
                    b = k % 2
                    nb = (k + 1) % 2
                    if k < 3:
                        gather_start(grp_cur[0], k + 1, nb)
                    elif k == 3:
                        idx_wait(srcgb, dstgb, semib)
                        gather_start(srcgb, 0, nb)
                    elif k < 7:
                        gather_start(srcgb, k - 3, nb)
                    else:
                        @pl.when(j0 + 8 < nch)
                        def _():
                            idx_wait(srcga, dstga, semia)
                            gather_start(srcga, 0, nb)
                    gather_wait(b)
                    pltpu.sync_copy(bufs[b], acc.at[grp_cur[1].at[k % 4]],
                                    add=True)
                    if k == 3:
                        @pl.when(j0 + 8 < nch)
                        def _():
                            idx_fill(j0 + 8, srcga, dstga, semia)

                @pl.when(j0 + 12 < nch)
                def _():
                    idx_fill(j0 + 12, srcgb, dstgb, semib)

            return carry

        lax.fori_loop(0, CHMAX // 8, body, 0)

        plsc.subcore_barrier()

        @pl.when(s < N // ROWS_OUT)
        def _():
            pltpu.sync_copy(acc.at[pl.ds(s * ROWS_OUT, ROWS_OUT)],
                            out_hbm.at[c, pl.ds(s * ROWS_OUT, ROWS_OUT)])

    return sc_aggregate


_sc_aggregate_d = _make_sc_aggregate(D)
_sc_aggregate_d2 = _make_sc_aggregate(D2)


BR = 1000


def _mm_body(x_ref, w_ref, o_ref):
    o_ref[...] = jnp.dot(x_ref[...], w_ref[...],
                         preferred_element_type=jnp.float32)


def _fuse_body(p_ref, b_ref, w_ref, o_ref):
    h = jnp.maximum(p_ref[0] + p_ref[1] + b_ref[...], 0.0)
    o_ref[...] = jnp.dot(h, w_ref[...], preferred_element_type=jnp.float32)


def _bias_body(p_ref, b_ref, o_ref):
    o_ref[...] = p_ref[0] + p_ref[1] + b_ref[...]


def _tc_matmul(x, w):
    return pl.pallas_call(
        _mm_body,
        grid=(N // BR,),
        in_specs=[pl.BlockSpec((BR, D), lambda i: (i, 0)),
                  pl.BlockSpec((D, D), lambda i: (0, 0))],
        out_specs=pl.BlockSpec((BR, D), lambda i: (i, 0)),
        out_shape=jax.ShapeDtypeStruct((N, D), jnp.float32),
    )(x, w)


def _tc_fused(p, b, w, dout):
    return pl.pallas_call(
        _fuse_body,
        grid=(N // BR,),
        in_specs=[pl.BlockSpec((2, BR, D), lambda i: (0, i, 0)),
                  pl.BlockSpec((1, D), lambda i: (0, 0)),
                  pl.BlockSpec((D, dout), lambda i: (0, 0))],
        out_specs=pl.BlockSpec((BR, dout), lambda i: (i, 0)),
        out_shape=jax.ShapeDtypeStruct((N, dout), jnp.float32),
    )(p, b, w)


def _tc_bias(p, b, dout):
    return pl.pallas_call(
        _bias_body,
        grid=(N // BR,),
        in_specs=[pl.BlockSpec((2, BR, dout), lambda i: (0, i, 0)),
                  pl.BlockSpec((1, dout), lambda i: (0, 0))],
        out_specs=pl.BlockSpec((BR, dout), lambda i: (i, 0)),
        out_shape=jax.ShapeDtypeStruct((N, dout), jnp.float32),
    )(p, b)


def kernel(features, edge_index, W0, b0, W1, b1, W2, b2):
    src = edge_index[0]
    dst = edge_index[1]
    pad = E_PAD - E
    srcs = jnp.concatenate(
        [src, jnp.zeros((pad,), jnp.int32)]).reshape(-1, CHUNK)
    dsts = jnp.concatenate(
        [dst, jnp.full((pad,), ACC_ROWS - 1, jnp.int32)]).reshape(-1, CHUNK)
    zeros = jnp.zeros((ZBLK, D), jnp.float32)
    zeros2 = jnp.zeros((ZBLK, D2), jnp.float32)

    a = _tc_matmul(features, W0)
    p = _sc_aggregate_d(a, srcs, dsts, zeros)
    c = _tc_fused(p, b0.reshape(1, D), W1, D)
    q = _sc_aggregate_d(c, srcs, dsts, zeros)
    w2p = jnp.pad(W2, ((0, 0), (0, D2 - NCLS)))
    h2 = _tc_fused(q, b1.reshape(1, D), w2p, D2)
    r = _sc_aggregate_d2(h2, srcs, dsts, zeros2)
    b2p = jnp.pad(b2, (0, D2 - NCLS)).reshape(1, D2)
    o = _tc_bias(r, b2p, D2)
    return o[:, :NCLS]


# speedup vs baseline: 1.0478x; 1.0160x over previous
import functools

import jax
import jax.numpy as jnp
from jax import lax
from jax.experimental import pallas as pl
from jax.experimental.pallas import tpu as pltpu
from jax.experimental.pallas import tpu_sc as plsc

N = 10000
E = 320000
D = 128
D2 = 64
NCLS = 40

NC = 2
NS = 16
CHUNK = 128
CH0 = 136
CH1 = 24
CHMAX = max(CH0, CH1)
G = 4
E_PAD = NS * (CH0 + CH1) * CHUNK
ZROWS = 640
ZBLK = 64
ACC_ROWS = NS * ZROWS
ROWS_OUT = 1000

_mesh = plsc.VectorSubcoreMesh(core_axis_name="c", subcore_axis_name="s")


def _make_sc_aggregate(d):
    @functools.partial(
        pl.kernel,
        mesh=_mesh,
        compiler_params=pltpu.CompilerParams(
            use_tc_tiling_on_sc=(d == D)),
        out_type=jax.ShapeDtypeStruct((NC, N, d), jnp.float32),
        scratch_types=[
            pltpu.VMEM((G, CHUNK), jnp.int32),
            pltpu.VMEM((G, CHUNK), jnp.int32),
            pltpu.VMEM((G, CHUNK), jnp.int32),
            pltpu.VMEM((G, CHUNK), jnp.int32),
            pltpu.VMEM((CHUNK, d), jnp.float32),
            pltpu.VMEM((CHUNK, d), jnp.float32),
            pltpu.SemaphoreType.DMA,
            pltpu.SemaphoreType.DMA,
            pltpu.SemaphoreType.DMA,
            pltpu.SemaphoreType.DMA,
            pltpu.VMEM_SHARED((ACC_ROWS, d), jnp.float32),
        ],
    )
    def sc_aggregate(h_hbm, srcs_hbm, dsts_hbm, zeros_hbm, out_hbm,
                     srcga, dstga, srcgb, dstgb, buf0, buf1,
                     semia, semib, semg0, semg1, acc):
        c = lax.axis_index("c")
        s = lax.axis_index("s")
        base = jnp.where(c == 0, s * CH0, NS * CH0 + s * CH1)
        nch = jnp.where(c == 0, CH0, CH1)

        bufs = (buf0, buf1)
        semg = (semg0, semg1)

        def idx_fill(grp_first_chunk, srcg, dstg, sem):
            pltpu.async_copy(srcs_hbm.at[pl.ds(base + grp_first_chunk, G)],
                             srcg, sem)
            pltpu.async_copy(dsts_hbm.at[pl.ds(base + grp_first_chunk, G)],
                             dstg, sem)

        def idx_wait(srcg, dstg, sem):
            pltpu.make_async_copy(srcs_hbm.at[pl.ds(0, G)], srcg, sem).wait()
            pltpu.make_async_copy(dsts_hbm.at[pl.ds(0, G)], dstg, sem).wait()

        def gather_start(srcg, k, b):
            pltpu.async_copy(h_hbm.at[srcg.at[k]], bufs[b], semg[b])

        def gather_wait(b):
            pltpu.make_async_copy(h_hbm.at[pl.ds(0, CHUNK)], bufs[b],
                                  semg[b]).wait()

        for k in range(ZROWS // ZBLK):
            pltpu.sync_copy(zeros_hbm,
                            acc.at[pl.ds(s * ZROWS + k * ZBLK, ZBLK)])
        plsc.subcore_barrier()

        @pl.when(nch > 0)
        def _():
            pltpu.sync_copy(srcs_hbm.at[pl.ds(base, G)], srcga)
            pltpu.sync_copy(dsts_hbm.at[pl.ds(base, G)], dstga)
            idx_fill(G, srcgb, dstgb, semib)
            gather_start(srcga, 0, 0)

        def body(u, carry):
            j0 = 8 * u

            @pl.when(j0 < nch)
            def _():
                for k in range(8):
                    grp_cur = (srcga, dstga) if k < 4 else (srcgb, dstgb)
                    b = k % 2
                    nb = (k + 1) % 2
                    if k < 3:
                        gather_start(grp_cur[0], k + 1, nb)
                    elif k == 3:
                        idx_wait(srcgb, dstgb, semib)
                        gather_start(srcgb, 0, nb)
                    elif k < 7:
                        gather_start(srcgb, k - 3, nb)
                    else:
                        @pl.when(j0 + 8 < nch)
                        def _():
                            idx_wait(srcga, dstga, semia)
                            gather_start(srcga, 0, nb)
                    gather_wait(b)
                    pltpu.sync_copy(bufs[b], acc.at[grp_cur[1].at[k % 4]],
                                    add=True)
                    if k == 3:
                        @pl.when(j0 + 8 < nch)
                        def _():
                            idx_fill(j0 + 8, srcga, dstga, semia)

                @pl.when(j0 + 12 < nch)
                def _():
                    idx_fill(j0 + 12, srcgb, dstgb, semib)

            return carry

        lax.fori_loop(0, CHMAX // 8, body, 0)

        plsc.subcore_barrier()

        @pl.when(s < N // ROWS_OUT)
        def _():
            pltpu.sync_copy(acc.at[pl.ds(s * ROWS_OUT, ROWS_OUT)],
                            out_hbm.at[c, pl.ds(s * ROWS_OUT, ROWS_OUT)])

    return sc_aggregate


_sc_aggregate_d = _make_sc_aggregate(D)
_sc_aggregate_d2 = _make_sc_aggregate(D2)


BR = 1000


def _mm_body(x_ref, w_ref, o_ref):
    o_ref[...] = jnp.dot(x_ref[...], w_ref[...],
                         preferred_element_type=jnp.float32)


def _fuse_body(p_ref, b_ref, w_ref, o_ref):
    h = jnp.maximum(p_ref[0] + p_ref[1] + b_ref[...], 0.0)
    o_ref[...] = jnp.dot(h, w_ref[...], preferred_element_type=jnp.float32)


def _bias_body(p_ref, b_ref, o_ref):
    o_ref[...] = p_ref[0] + p_ref[1] + b_ref[...]


def _tc_matmul(x, w):
    return pl.pallas_call(
        _mm_body,
        grid=(N // BR,),
        in_specs=[pl.BlockSpec((BR, D), lambda i: (i, 0)),
                  pl.BlockSpec((D, D), lambda i: (0, 0))],
        out_specs=pl.BlockSpec((BR, D), lambda i: (i, 0)),
        out_shape=jax.ShapeDtypeStruct((N, D), jnp.float32),
    )(x, w)


def _tc_fused(p, b, w, dout):
    return pl.pallas_call(
        _fuse_body,
        grid=(N // BR,),
        in_specs=[pl.BlockSpec((2, BR, D), lambda i: (0, i, 0)),
                  pl.BlockSpec((1, D), lambda i: (0, 0)),
                  pl.BlockSpec((D, dout), lambda i: (0, 0))],
        out_specs=pl.BlockSpec((BR, dout), lambda i: (i, 0)),
        out_shape=jax.ShapeDtypeStruct((N, dout), jnp.float32),
    )(p, b, w)


def _tc_bias(p, b, dout):
    return pl.pallas_call(
        _bias_body,
        grid=(N // BR,),
        in_specs=[pl.BlockSpec((2, BR, dout), lambda i: (0, i, 0)),
                  pl.BlockSpec((1, dout), lambda i: (0, 0))],
        out_specs=pl.BlockSpec((BR, dout), lambda i: (i, 0)),
        out_shape=jax.ShapeDtypeStruct((N, dout), jnp.float32),
    )(p, b)


def kernel(features, edge_index, W0, b0, W1, b1, W2, b2):
    src = edge_index[0]
    dst = edge_index[1]
    pad = E_PAD - E
    srcs = jnp.concatenate(
        [src, jnp.zeros((pad,), jnp.int32)]).reshape(-1, CHUNK)
    dsts = jnp.concatenate(
        [dst, jnp.full((pad,), ACC_ROWS - 1, jnp.int32)]).reshape(-1, CHUNK)
    zeros = jnp.zeros((ZBLK, D), jnp.float32)
    zeros2 = jnp.zeros((ZBLK, D2), jnp.float32)

    a = _tc_matmul(features, W0)
    p = _sc_aggregate_d(a, srcs, dsts, zeros)
    c = _tc_fused(p, b0.reshape(1, D), W1, D)
    q = _sc_aggregate_d(c, srcs, dsts, zeros)
    w2p = jnp.pad(W2, ((0, 0), (0, D2 - NCLS)))
    h2 = _tc_fused(q, b1.reshape(1, D), w2p, D2)
    r = _sc_aggregate_d2(h2, srcs, dsts, zeros2)
    b2p = jnp.pad(b2, (0, D2 - NCLS)).reshape(1, D2)
    o = _tc_bias(r, b2p, D2)
    return o[:, :NCLS]


# speedup vs baseline: 1.1529x; 1.1003x over previous
import functools

import jax
import jax.numpy as jnp
from jax import lax
from jax.experimental import pallas as pl
from jax.experimental.pallas import tpu as pltpu
from jax.experimental.pallas import tpu_sc as plsc

N = 10000
E = 320000
D = 128
D2 = 64
NCLS = 40

NC = 2
NS = 16
CHUNK = 128
CH0 = 144
CH1 = 16
CHMAX = max(CH0, CH1)
G = 4
E_PAD = NS * (CH0 + CH1) * CHUNK
ZROWS = 640
ZBLK = 64
ACC_ROWS = NS * ZROWS
ROWS_OUT = 1000

_mesh = plsc.VectorSubcoreMesh(core_axis_name="c", subcore_axis_name="s")


def _make_sc_aggregate(d):
    @functools.partial(
        pl.kernel,
        mesh=_mesh,
        compiler_params=pltpu.CompilerParams(
            use_tc_tiling_on_sc=(d == D)),
        out_type=jax.ShapeDtypeStruct((NC, N, d), jnp.float32),
        scratch_types=[
            pltpu.VMEM((G, CHUNK), jnp.int32),
            pltpu.VMEM((G, CHUNK), jnp.int32),
            pltpu.VMEM((G, CHUNK), jnp.int32),
            pltpu.VMEM((G, CHUNK), jnp.int32),
            pltpu.VMEM((CHUNK, d), jnp.float32),
            pltpu.VMEM((CHUNK, d), jnp.float32),
            pltpu.SemaphoreType.DMA,
            pltpu.SemaphoreType.DMA,
            pltpu.SemaphoreType.DMA,
            pltpu.SemaphoreType.DMA,
            pltpu.VMEM_SHARED((ACC_ROWS, d), jnp.float32),
        ],
    )
    def sc_aggregate(h_hbm, srcs_hbm, dsts_hbm, zeros_hbm, out_hbm,
                     srcga, dstga, srcgb, dstgb, buf0, buf1,
                     semia, semib, semg0, semg1, acc):
        c = lax.axis_index("c")
        s = lax.axis_index("s")
        base = jnp.where(c == 0, s * CH0, NS * CH0 + s * CH1)
        nch = jnp.where(c == 0, CH0, CH1)

        bufs = (buf0, buf1)
        semg = (semg0, semg1)

        def idx_fill(grp_first_chunk, srcg, dstg, sem):
            pltpu.async_copy(srcs_hbm.at[pl.ds(base + grp_first_chunk, G)],
                             srcg, sem)
            pltpu.async_copy(dsts_hbm.at[pl.ds(base + grp_first_chunk, G)],
                             dstg, sem)

        def idx_wait(srcg, dstg, sem):
            pltpu.make_async_copy(srcs_hbm.at[pl.ds(0, G)], srcg, sem).wait()
            pltpu.make_async_copy(dsts_hbm.at[pl.ds(0, G)], dstg, sem).wait()

        def gather_start(srcg, k, b):
            pltpu.async_copy(h_hbm.at[srcg.at[k]], bufs[b], semg[b])

        def gather_wait(b):
            pltpu.make_async_copy(h_hbm.at[pl.ds(0, CHUNK)], bufs[b],
                                  semg[b]).wait()

        for k in range(ZROWS // ZBLK):
            pltpu.sync_copy(zeros_hbm,
                            acc.at[pl.ds(s * ZROWS + k * ZBLK, ZBLK)])
        plsc.subcore_barrier()

        @pl.when(nch > 0)
        def _():
            pltpu.sync_copy(srcs_hbm.at[pl.ds(base, G)], srcga)
            pltpu.sync_copy(dsts_hbm.at[pl.ds(base, G)], dstga)
            idx_fill(G, srcgb, dstgb, semib)
            gather_start(srcga, 0, 0)

        def body(u, carry):
            j0 = 8 * u

            @pl.when(j0 < nch)
            def _():
                for k in range(8):
                    grp_cur = (srcga, dstga) if k < 4 else (srcgb, dstgb)
                    b = k % 2
                    nb = (k + 1) % 2
                    if k < 3:
                        gather_start(grp_cur[0], k + 1, nb)
                    elif k == 3:
                        idx_wait(srcgb, dstgb, semib)
                        gather_start(srcgb, 0, nb)
                    elif k < 7:
                        gather_start(srcgb, k - 3, nb)
                    else:
                        @pl.when(j0 + 8 < nch)
                        def _():
                            idx_wait(srcga, dstga, semia)
                            gather_start(srcga, 0, nb)
                    gather_wait(b)
                    pltpu.sync_copy(bufs[b], acc.at[grp_cur[1].at[k % 4]],
                                    add=True)
                    if k == 3:
                        @pl.when(j0 + 8 < nch)
                        def _():
                            idx_fill(j0 + 8, srcga, dstga, semia)

                @pl.when(j0 + 12 < nch)
                def _():
                    idx_fill(j0 + 12, srcgb, dstgb, semib)

            return carry

        lax.fori_loop(0, CHMAX // 8, body, 0)

        plsc.subcore_barrier()

        @pl.when(s < N // ROWS_OUT)
        def _():
            pltpu.sync_copy(acc.at[pl.ds(s * ROWS_OUT, ROWS_OUT)],
                            out_hbm.at[c, pl.ds(s * ROWS_OUT, ROWS_OUT)])

    return sc_aggregate


_sc_aggregate_d = _make_sc_aggregate(D)
_sc_aggregate_d2 = _make_sc_aggregate(D2)


BR = 1000


def _mm_body(x_ref, w_ref, o_ref):
    o_ref[...] = jnp.dot(x_ref[...], w_ref[...],
                         preferred_element_type=jnp.float32)


def _fuse_body(p_ref, b_ref, w_ref, o_ref):
    h = jnp.maximum(p_ref[0] + p_ref[1] + b_ref[...], 0.0)
    o_ref[...] = jnp.dot(h, w_ref[...], preferred_element_type=jnp.float32)


def _bias_body(p_ref, b_ref, o_ref):
    o_ref[...] = p_ref[0] + p_ref[1] + b_ref[...]


def _tc_matmul(x, w):
    return pl.pallas_call(
        _mm_body,
        grid=(N // BR,),
        in_specs=[pl.BlockSpec((BR, D), lambda i: (i, 0)),
                  pl.BlockSpec((D, D), lambda i: (0, 0))],
        out_specs=pl.BlockSpec((BR, D), lambda i: (i, 0)),
        out_shape=jax.ShapeDtypeStruct((N, D), jnp.float32),
    )(x, w)


def _tc_fused(p, b, w, dout):
    return pl.pallas_call(
        _fuse_body,
        grid=(N // BR,),
        in_specs=[pl.BlockSpec((2, BR, D), lambda i: (0, i, 0)),
                  pl.BlockSpec((1, D), lambda i: (0, 0)),
                  pl.BlockSpec((D, dout), lambda i: (0, 0))],
        out_specs=pl.BlockSpec((BR, dout), lambda i: (i, 0)),
        out_shape=jax.ShapeDtypeStruct((N, dout), jnp.float32),
    )(p, b, w)


def _tc_bias(p, b, dout):
    return pl.pallas_call(
        _bias_body,
        grid=(N // BR,),
        in_specs=[pl.BlockSpec((2, BR, dout), lambda i: (0, i, 0)),
                  pl.BlockSpec((1, dout), lambda i: (0, 0))],
        out_specs=pl.BlockSpec((BR, dout), lambda i: (i, 0)),
        out_shape=jax.ShapeDtypeStruct((N, dout), jnp.float32),
    )(p, b)


def kernel(features, edge_index, W0, b0, W1, b1, W2, b2):
    src = edge_index[0]
    dst = edge_index[1]
    pad = E_PAD - E
    srcs = jnp.concatenate(
        [src, jnp.zeros((pad,), jnp.int32)]).reshape(-1, CHUNK)
    dsts = jnp.concatenate(
        [dst, jnp.full((pad,), ACC_ROWS - 1, jnp.int32)]).reshape(-1, CHUNK)
    zeros = jnp.zeros((ZBLK, D), jnp.float32)
    zeros2 = jnp.zeros((ZBLK, D2), jnp.float32)

    a = _tc_matmul(features, W0)
    p = _sc_aggregate_d(a, srcs, dsts, zeros)
    c = _tc_fused(p, b0.reshape(1, D), W1, D)
    q = _sc_aggregate_d(c, srcs, dsts, zeros)
    w2p = jnp.pad(W2, ((0, 0), (0, D2 - NCLS)))
    h2 = _tc_fused(q, b1.reshape(1, D), w2p, D2)
    r = _sc_aggregate_d2(h2, srcs, dsts, zeros2)
    b2p = jnp.pad(b2, (0, D2 - NCLS)).reshape(1, D2)
    o = _tc_bias(r, b2p, D2)
    return o[:, :NCLS]


# speedup vs baseline: 1.1581x; 1.0045x over previous
import functools

import jax
import jax.numpy as jnp
from jax import lax
from jax.experimental import pallas as pl
from jax.experimental.pallas import tpu as pltpu
from jax.experimental.pallas import tpu_sc as plsc

N = 10000
E = 320000
D = 128
D2 = 64
NCLS = 40

NC = 2
NS = 16
CHUNK = 128
CH0 = 152
CH1 = 8
CHMAX = max(CH0, CH1)
G = 4
E_PAD = NS * (CH0 + CH1) * CHUNK
ZROWS = 640
ZBLK = 64
ACC_ROWS = NS * ZROWS
ROWS_OUT = 1000

_mesh = plsc.VectorSubcoreMesh(core_axis_name="c", subcore_axis_name="s")


def _make_sc_aggregate(d):
    @functools.partial(
        pl.kernel,
        mesh=_mesh,
        compiler_params=pltpu.CompilerParams(
            use_tc_tiling_on_sc=(d == D)),
        out_type=jax.ShapeDtypeStruct((NC, N, d), jnp.float32),
        scratch_types=[
            pltpu.VMEM((G, CHUNK), jnp.int32),
            pltpu.VMEM((G, CHUNK), jnp.int32),
            pltpu.VMEM((G, CHUNK), jnp.int32),
            pltpu.VMEM((G, CHUNK), jnp.int32),
            pltpu.VMEM((CHUNK, d), jnp.float32),
            pltpu.VMEM((CHUNK, d), jnp.float32),
            pltpu.SemaphoreType.DMA,
            pltpu.SemaphoreType.DMA,
            pltpu.SemaphoreType.DMA,
            pltpu.SemaphoreType.DMA,
            pltpu.VMEM_SHARED((ACC_ROWS, d), jnp.float32),
        ],
    )
    def sc_aggregate(h_hbm, srcs_hbm, dsts_hbm, zeros_hbm, out_hbm,
                     srcga, dstga, srcgb, dstgb, buf0, buf1,
                     semia, semib, semg0, semg1, acc):
        c = lax.axis_index("c")
        s = lax.axis_index("s")
        base = jnp.where(c == 0, s * CH0, NS * CH0 + s * CH1)
        nch = jnp.where(c == 0, CH0, CH1)

        bufs = (buf0, buf1)
        semg = (semg0, semg1)

        def idx_fill(grp_first_chunk, srcg, dstg, sem):
            pltpu.async_copy(srcs_hbm.at[pl.ds(base + grp_first_chunk, G)],
                             srcg, sem)
            pltpu.async_copy(dsts_hbm.at[pl.ds(base + grp_first_chunk, G)],
                             dstg, sem)

        def idx_wait(srcg, dstg, sem):
            pltpu.make_async_copy(srcs_hbm.at[pl.ds(0, G)], srcg, sem).wait()
            pltpu.make_async_copy(dsts_hbm.at[pl.ds(0, G)], dstg, sem).wait()

        def gather_start(srcg, k, b):
            pltpu.async_copy(h_hbm.at[srcg.at[k]], bufs[b], semg[b])

        def gather_wait(b):
            pltpu.make_async_copy(h_hbm.at[pl.ds(0, CHUNK)], bufs[b],
                                  semg[b]).wait()

        for k in range(ZROWS // ZBLK):
            pltpu.sync_copy(zeros_hbm,
                            acc.at[pl.ds(s * ZROWS + k * ZBLK, ZBLK)])
        plsc.subcore_barrier()

        @pl.when(nch > 0)
        def _():
            pltpu.sync_copy(srcs_hbm.at[pl.ds(base, G)], srcga)
            pltpu.sync_copy(dsts_hbm.at[pl.ds(base, G)], dstga)
            idx_fill(G, srcgb, dstgb, semib)
            gather_start(srcga, 0, 0)

        def body(u, carry):
            j0 = 8 * u

            @pl.when(j0 < nch)
            def _():
                for k in range(8):
                    grp_cur = (srcga, dstga) if k < 4 else (srcgb, dstgb)
                    b = k % 2
                    nb = (k + 1) % 2
                    if k < 3:
                        gather_start(grp_cur[0], k + 1, nb)
                    elif k == 3:
                        idx_wait(srcgb, dstgb, semib)
                        gather_start(srcgb, 0, nb)
                    elif k < 7:
                        gather_start(srcgb, k - 3, nb)
                    else:
                        @pl.when(j0 + 8 < nch)
                        def _():
                            idx_wait(srcga, dstga, semia)
                            gather_start(srcga, 0, nb)
                    gather_wait(b)
                    pltpu.sync_copy(bufs[b], acc.at[grp_cur[1].at[k % 4]],
                                    add=True)
                    if k == 3:
                        @pl.when(j0 + 8 < nch)
                        def _():
                            idx_fill(j0 + 8, srcga, dstga, semia)

                @pl.when(j0 + 12 < nch)
                def _():
                    idx_fill(j0 + 12, srcgb, dstgb, semib)

            return carry

        lax.fori_loop(0, CHMAX // 8, body, 0)

        plsc.subcore_barrier()

        @pl.when(s < N // ROWS_OUT)
        def _():
            pltpu.sync_copy(acc.at[pl.ds(s * ROWS_OUT, ROWS_OUT)],
                            out_hbm.at[c, pl.ds(s * ROWS_OUT, ROWS_OUT)])

    return sc_aggregate


_sc_aggregate_d = _make_sc_aggregate(D)
_sc_aggregate_d2 = _make_sc_aggregate(D2)


BR = 1000


def _mm_body(x_ref, w_ref, o_ref):
    o_ref[...] = jnp.dot(x_ref[...], w_ref[...],
                         preferred_element_type=jnp.float32)


def _fuse_body(p_ref, b_ref, w_ref, o_ref):
    h = jnp.maximum(p_ref[0] + p_ref[1] + b_ref[...], 0.0)
    o_ref[...] = jnp.dot(h, w_ref[...], preferred_element_type=jnp.float32)


def _bias_body(p_ref, b_ref, o_ref):
    o_ref[...] = p_ref[0] + p_ref[1] + b_ref[...]


def _tc_matmul(x, w):
    return pl.pallas_call(
        _mm_body,
        grid=(N // BR,),
        in_specs=[pl.BlockSpec((BR, D), lambda i: (i, 0)),
                  pl.BlockSpec((D, D), lambda i: (0, 0))],
        out_specs=pl.BlockSpec((BR, D), lambda i: (i, 0)),
        out_shape=jax.ShapeDtypeStruct((N, D), jnp.float32),
    )(x, w)


def _tc_fused(p, b, w, dout):
    return pl.pallas_call(
        _fuse_body,
        grid=(N // BR,),
        in_specs=[pl.BlockSpec((2, BR, D), lambda i: (0, i, 0)),
                  pl.BlockSpec((1, D), lambda i: (0, 0)),
                  pl.BlockSpec((D, dout), lambda i: (0, 0))],
        out_specs=pl.BlockSpec((BR, dout), lambda i: (i, 0)),
        out_shape=jax.ShapeDtypeStruct((N, dout), jnp.float32),
    )(p, b, w)


def _tc_bias(p, b, dout):
    return pl.pallas_call(
        _bias_body,
        grid=(N // BR,),
        in_specs=[pl.BlockSpec((2, BR, dout), lambda i: (0, i, 0)),
                  pl.BlockSpec((1, dout), lambda i: (0, 0))],
        out_specs=pl.BlockSpec((BR, dout), lambda i: (i, 0)),
        out_shape=jax.ShapeDtypeStruct((N, dout), jnp.float32),
    )(p, b)


def kernel(features, edge_index, W0, b0, W1, b1, W2, b2):
    src = edge_index[0]
    dst = edge_index[1]
    pad = E_PAD - E
    srcs = jnp.concatenate(
        [src, jnp.zeros((pad,), jnp.int32)]).reshape(-1, CHUNK)
    dsts = jnp.concatenate(
        [dst, jnp.full((pad,), ACC_ROWS - 1, jnp.int32)]).reshape(-1, CHUNK)
    zeros = jnp.zeros((ZBLK, D), jnp.float32)
    zeros2 = jnp.zeros((ZBLK, D2), jnp.float32)

    a = _tc_matmul(features, W0)
    p = _sc_aggregate_d(a, srcs, dsts, zeros)
    c = _tc_fused(p, b0.reshape(1, D), W1, D)
    q = _sc_aggregate_d(c, srcs, dsts, zeros)
    w2p = jnp.pad(W2, ((0, 0), (0, D2 - NCLS)))
    h2 = _tc_fused(q, b1.reshape(1, D), w2p, D2)
    r = _sc_aggregate_d2(h2, srcs, dsts, zeros2)
    b2p = jnp.pad(b2, (0, D2 - NCLS)).reshape(1, D2)
    o = _tc_bias(r, b2p, D2)
    return o[:, :NCLS]
